# baseline (speedup 1.0000x reference)
import jax
import jax.numpy as jnp
from jax import lax
from jax.experimental import pallas as pl
from jax.experimental.pallas import tpu as pltpu
from jax.experimental.pallas import tpu_sc as plsc

N = 10000
E = 160000
NC, NS, L = 2, 16, 16
ER = 1280
EP = ER * 128
NACC = 10240
RPT = NACC // NS
RB = 400
GRID = N // RB


def _make_segsum(col_split):
    mesh = plsc.VectorSubcoreMesh(core_axis_name="c", subcore_axis_name="s")
    outs = [jax.ShapeDtypeStruct((NACC, 128), jnp.float32),
            jax.ShapeDtypeStruct((NACC, 128), jnp.float32)]
    if col_split:
        outs += [jax.ShapeDtypeStruct((NACC,), jnp.float32),
                 jax.ShapeDtypeStruct((NACC,), jnp.float32)]
    scratch = [
        pltpu.VMEM_SHARED((NACC, 128), jnp.float32),
        pltpu.VMEM((2, 8, 128), jnp.int32),
        pltpu.VMEM((2, 8, 128), jnp.int32),
        pltpu.VMEM((128, 128), jnp.float32),
        pltpu.VMEM((128, 128), jnp.float32),
        pltpu.SemaphoreType.DMA,
        pltpu.SemaphoreType.DMA,
        pltpu.SemaphoreType.DMA,
        pltpu.SemaphoreType.DMA,
        pltpu.SemaphoreType.DMA,
    ]
    if col_split:
        scratch += [
            pltpu.VMEM_SHARED((NACC,), jnp.float32),
            pltpu.VMEM((128,), jnp.float32),
            pltpu.VMEM((RPT,), jnp.float32),
        ]

    def body(*refs):
        if col_split:
            (t0, t1, srcr, dstr, o0, o1, degoutA, degoutB,
             acc, sidx, didx, rows_a, rows_b,
             gsem, ssem, dsem, isem, zsem, degacc, ones_v, zdeg_v) = refs
        else:
            (t0, t1, srcr, dstr, o0, o1,
             acc, sidx, didx, rows_a, rows_b,
             gsem, ssem, dsem, isem, zsem) = refs
        c = lax.axis_index("c")
        s = lax.axis_index("s")
        rowbuf = [rows_a, rows_b]

        def zrow(i, carry):
            for j in range(128 // L):
                rows_a[i, pl.ds(j * L, L)] = jnp.zeros((L,), jnp.float32)
            return carry
        lax.fori_loop(0, 128, zrow, 0)
        for k in range(RPT // 128):
            pltpu.async_copy(rows_a, acc.at[pl.ds(s * RPT + k * 128, 128), :],
                             zsem)
        if col_split:
            def zdrow(i, carry):
                zdeg_v[pl.ds(i * L, L)] = jnp.zeros((L,), jnp.float32)
                return carry
            lax.fori_loop(0, RPT // L, zdrow, 0)
            for j in range(128 // L):
                ones_v[pl.ds(j * L, L)] = jnp.full((L,), 1.0, jnp.float32)
            pltpu.sync_copy(zdeg_v, degacc.at[pl.ds(s * RPT, RPT)])
        for k in range(RPT // 128):
            pltpu.make_async_copy(rows_a,
                                  acc.at[pl.ds(s * RPT + k * 128, 128), :],
                                  zsem).wait()
        plsc.subcore_barrier()

        if col_split:
            nrow = ER // NS
            row0 = s * nrow
            dflags = [[r < nrow // 2 for r in range(nrow)],
                      [r >= nrow // 2 for r in range(nrow)]]
        else:
            nrow = ER // NS // NC
            dflags = None

        def run(tbl, row0, dflag):
            nchunks = nrow // 8

            def idx_copies(g, slot):
                base = row0 + g * 8
                return (pltpu.make_async_copy(srcr.at[pl.ds(base, 8), :],
                                              sidx.at[slot], isem),
                        pltpu.make_async_copy(dstr.at[pl.ds(base, 8), :],
                                              didx.at[slot], isem))

            pltpu.sync_copy(srcr.at[pl.ds(row0, 8), :], sidx.at[0])
            pltpu.sync_copy(dstr.at[pl.ds(row0, 8), :], didx.at[0])
            if nchunks > 1:
                for d in idx_copies(1, 1):
                    d.start()
            pltpu.async_copy(tbl.at[sidx.at[0, 0]], rows_a, gsem)
            for r in range(nrow):
                gch, i = divmod(r, 8)
                sl = gch % 2
                b = r % 2
                r1 = r + 1
                if r1 < nrow:
                    g1, i1 = divmod(r1, 8)
                    sl1 = g1 % 2
                    if i1 == 0:
                        for d in idx_copies(g1, sl1):
                            d.wait()
                    if r1 >= 2:
                        pltpu.make_async_copy(rowbuf[r1 % 2],
                                              acc.at[didx.at[sl1, i1]],
                                              ssem).wait()
                        if dflag and dflag[r - 1]:
                            pltpu.make_async_copy(ones_v,
                                                  degacc.at[didx.at[sl1, i1]],
                                                  dsem).wait()
                    pltpu.async_copy(tbl.at[sidx.at[sl1, i1]],
                                     rowbuf[r1 % 2], gsem)
                    if i == 2 and gch >= 1 and gch + 1 < nchunks:
                        for d in idx_copies(gch + 1, (gch + 1) % 2):
                            d.start()
                pltpu.make_async_copy(tbl.at[sidx.at[sl, i]], rowbuf[b],
                                      gsem).wait()
                pltpu.async_copy(rowbuf[b], acc.at[didx.at[sl, i]], ssem,
                                 add=True)
                if dflag and dflag[r]:
                    pltpu.async_copy(ones_v, degacc.at[didx.at[sl, i]], dsem,
                                     add=True)
            for rr in (nrow - 2, nrow - 1):
                gch, i = divmod(rr, 8)
                pltpu.make_async_copy(rowbuf[rr % 2],
                                      acc.at[didx.at[gch % 2, i]],
                                      ssem).wait()
                if dflag and dflag[rr]:
                    pltpu.make_async_copy(ones_v,
                                          degacc.at[didx.at[gch % 2, i]],
                                          dsem).wait()

        if col_split:
            @pl.when(c == 0)
            def _():
                run(t0, s * nrow, dflags[0])

            @pl.when(c == 1)
            def _():
                run(t1, s * nrow, dflags[1])
        else:
            @pl.when(c == 1)
            def _():
                run(t1, s * (2 * nrow) + nrow, None)

            @pl.when(c == 0)
            def _():
                run(t0, s * (2 * nrow), None)

        plsc.subcore_barrier()

        @pl.when(c == 0)
        def _():
            pltpu.sync_copy(acc.at[pl.ds(s * RPT, RPT), :],
                            o0.at[pl.ds(s * RPT, RPT), :])
            if col_split:
                pltpu.sync_copy(degacc.at[pl.ds(s * RPT, RPT)],
                                degoutA.at[pl.ds(s * RPT, RPT)])

        @pl.when(c == 1)
        def _():
            pltpu.sync_copy(acc.at[pl.ds(s * RPT, RPT), :],
                            o1.at[pl.ds(s * RPT, RPT), :])
            if col_split:
                pltpu.sync_copy(degacc.at[pl.ds(s * RPT, RPT)],
                                degoutB.at[pl.ds(s * RPT, RPT)])

    return pl.kernel(body, out_type=tuple(outs), mesh=mesh,
                     scratch_types=scratch)


_segsum_l1 = _make_segsum(True)
_segsum_l2 = _make_segsum(False)

_DN = (((1,), (1,)), ((), ()))


def _dot(a, w):
    return lax.dot_general(a, w, _DN, precision=lax.Precision.HIGHEST,
                           preferred_element_type=jnp.float32)


def _tc0_body(xr, w1r, out):
    out[...] = _dot(xr[...], w1r[...])


def _tc1_body(a0, a1, dA, dB, xw, w1l, b1r, w2l, ho, p2o, p2o2):
    inv = 1.0 / jnp.maximum(dA[...] + dB[...], 1.0)
    m0 = a0[...] * inv
    m1 = a1[...] * inv
    w1l_v = w1l[...]
    h = (_dot(m0, w1l_v[:, :128]) + _dot(m1, w1l_v[:, 128:])
         + xw[...] + b1r[...])
    h = jnp.maximum(h, 0.0)
    ho[...] = h
    p2 = _dot(h, w2l[...])
    p2o[...] = p2
    p2o2[...] = p2


def _tc1b_body(h, w2r, out):
    out[...] = _dot(h[...], w2r[...])


def _tc2_body(aA, aB, dA, dB, r2, b2r, out):
    inv = 1.0 / jnp.maximum(dA[...] + dB[...], 1.0)
    z = (aA[...] + aB[...]) * inv + b2r[...] + r2[...]
    m = jnp.max(z, axis=1, keepdims=True)
    lse = jnp.log(jnp.sum(jnp.exp(z - m), axis=1, keepdims=True))
    out[...] = z - m - lse


def _row_spec(d):
    return pl.BlockSpec((RB, d), lambda i: (i, 0))


def _full_spec(r, d):
    return pl.BlockSpec((r, d), lambda i: (0, 0))


_tc0 = pl.pallas_call(
    _tc0_body,
    grid=(GRID,),
    in_specs=[_row_spec(256), _full_spec(256, 256)],
    out_specs=_row_spec(256),
    out_shape=jax.ShapeDtypeStruct((N, 256), jnp.float32),
)

_tc1 = pl.pallas_call(
    _tc1_body,
    grid=(GRID,),
    in_specs=[_row_spec(128), _row_spec(128), _row_spec(1), _row_spec(1),
              _row_spec(256), _full_spec(256, 256), _full_spec(1, 256),
              _full_spec(128, 256)],
    out_specs=[_row_spec(256), _row_spec(128), _row_spec(128)],
    out_shape=[jax.ShapeDtypeStruct((N, 256), jnp.float32),
               jax.ShapeDtypeStruct((N, 128), jnp.float32),
               jax.ShapeDtypeStruct((N, 128), jnp.float32)],
)

_tc1b = pl.pallas_call(
    _tc1b_body,
    grid=(GRID,),
    in_specs=[_row_spec(256), _full_spec(128, 256)],
    out_specs=_row_spec(128),
    out_shape=jax.ShapeDtypeStruct((N, 128), jnp.float32),
)

_tc2 = pl.pallas_call(
    _tc2_body,
    grid=(GRID,),
    in_specs=[_row_spec(128), _row_spec(128), _row_spec(1), _row_spec(1),
              _row_spec(128), _full_spec(1, 128)],
    out_specs=_row_spec(128),
    out_shape=jax.ShapeDtypeStruct((N, 128), jnp.float32),
)


@jax.jit
def kernel(x, edge_index, W1_l, b1_l, W1_r, W2_l, b2_l, W2_r):
    src = edge_index[0].astype(jnp.int32)
    dst = edge_index[1].astype(jnp.int32)
    srcp = jnp.concatenate([src, jnp.zeros((EP - E,), jnp.int32)]).reshape(ER, 128)
    padd = N + (jnp.arange(EP - E, dtype=jnp.int32) % (NACC - N))
    dstp = jnp.concatenate([dst, padd]).reshape(ER, 128)
    x0 = x[:, :128]
    x1 = x[:, 128:]

    xw = _tc0(x, W1_r)
    agg0, agg1, degA, degB = _segsum_l1(x0, x1, srcp, dstp)
    dA = degA[:N].reshape(N, 1)
    dB = degB[:N].reshape(N, 1)
    h, p2a, p2b = _tc1(agg0, agg1, dA, dB, xw, W1_l, b1_l.reshape(1, -1),
                       W2_l)
    r2 = _tc1b(h, W2_r)
    a2A, a2B = _segsum_l2(p2a, p2b, srcp, dstp)
    return _tc2(a2A, a2B, dA, dB, r2, b2_l.reshape(1, -1))


# speedup vs baseline: 1.1443x; 1.1443x over previous
import jax
import jax.numpy as jnp
from jax import lax
from jax.experimental import pallas as pl
from jax.experimental.pallas import tpu as pltpu
from jax.experimental.pallas import tpu_sc as plsc

N = 10000
E = 160000
NC, NS, L = 2, 16, 16
ER = 1280
EP = ER * 128
NACC = 10240
RPT = NACC // NS
RB = 400
GRID = N // RB


def _make_segsum(col_split):
    mesh = plsc.VectorSubcoreMesh(core_axis_name="c", subcore_axis_name="s")
    outs = [jax.ShapeDtypeStruct((NACC, 128), jnp.float32),
            jax.ShapeDtypeStruct((NACC, 128), jnp.float32)]
    if col_split:
        outs += [jax.ShapeDtypeStruct((NACC,), jnp.float32),
                 jax.ShapeDtypeStruct((NACC,), jnp.float32)]
    scratch = [
        pltpu.VMEM_SHARED((NACC, 128), jnp.float32),
        pltpu.VMEM((2, 8, 128), jnp.int32),
        pltpu.VMEM((2, 8, 128), jnp.int32),
        pltpu.VMEM((128, 128), jnp.float32),
        pltpu.VMEM((128, 128), jnp.float32),
        pltpu.SemaphoreType.DMA,
        pltpu.SemaphoreType.DMA,
        pltpu.SemaphoreType.DMA,
        pltpu.SemaphoreType.DMA,
        pltpu.SemaphoreType.DMA,
    ]
    if col_split:
        scratch += [
            pltpu.VMEM_SHARED((NACC,), jnp.float32),
            pltpu.VMEM((128,), jnp.float32),
            pltpu.VMEM((RPT,), jnp.float32),
        ]

    def body(*refs):
        if col_split:
            (t0, t1, srcr, dstr, o0, o1, degoutA, degoutB,
             acc, sidx, didx, rows_a, rows_b,
             gsem, ssem, dsem, isem, zsem, degacc, ones_v, zdeg_v) = refs
        else:
            (t0, t1, srcr, dstr, o0, o1,
             acc, sidx, didx, rows_a, rows_b,
             gsem, ssem, dsem, isem, zsem) = refs
        c = lax.axis_index("c")
        s = lax.axis_index("s")
        rowbuf = [rows_a, rows_b]

        def zrow(i, carry):
            for j in range(128 // L):
                rows_a[i, pl.ds(j * L, L)] = jnp.zeros((L,), jnp.float32)
            return carry
        lax.fori_loop(0, 128, zrow, 0)
        for k in range(RPT // 128):
            pltpu.async_copy(rows_a, acc.at[pl.ds(s * RPT + k * 128, 128), :],
                             zsem)
        if col_split:
            def zdrow(i, carry):
                zdeg_v[pl.ds(i * L, L)] = jnp.zeros((L,), jnp.float32)
                return carry
            lax.fori_loop(0, RPT // L, zdrow, 0)
            for j in range(128 // L):
                ones_v[pl.ds(j * L, L)] = jnp.full((L,), 1.0, jnp.float32)
            pltpu.sync_copy(zdeg_v, degacc.at[pl.ds(s * RPT, RPT)])
        for k in range(RPT // 128):
            pltpu.make_async_copy(rows_a,
                                  acc.at[pl.ds(s * RPT + k * 128, 128), :],
                                  zsem).wait()
        plsc.subcore_barrier()

        if col_split:
            nrow = ER // NS
            row0 = s * nrow
            dflags = [[r < nrow // 2 for r in range(nrow)],
                      [r >= nrow // 2 for r in range(nrow)]]
        else:
            nrow = ER // NS // NC
            dflags = None

        def run(tbl, row0, dflag, nrow_n=None):
            nr = nrow if nrow_n is None else nrow_n
            nchunks = nr // 8

            def idx_copies(g, slot):
                base = row0 + g * 8
                return (pltpu.make_async_copy(srcr.at[pl.ds(base, 8), :],
                                              sidx.at[slot], isem),
                        pltpu.make_async_copy(dstr.at[pl.ds(base, 8), :],
                                              didx.at[slot], isem))

            pltpu.sync_copy(srcr.at[pl.ds(row0, 8), :], sidx.at[0])
            pltpu.sync_copy(dstr.at[pl.ds(row0, 8), :], didx.at[0])
            if nchunks > 1:
                for d in idx_copies(1, 1):
                    d.start()
            pltpu.async_copy(tbl.at[sidx.at[0, 0]], rows_a, gsem)
            for r in range(nr):
                gch, i = divmod(r, 8)
                sl = gch % 2
                b = r % 2
                r1 = r + 1
                if r1 < nr:
                    g1, i1 = divmod(r1, 8)
                    sl1 = g1 % 2
                    if i1 == 0:
                        for d in idx_copies(g1, sl1):
                            d.wait()
                    if r1 >= 2:
                        pltpu.make_async_copy(rowbuf[r1 % 2],
                                              acc.at[didx.at[sl1, i1]],
                                              ssem).wait()
                        if dflag and dflag[r - 1]:
                            pltpu.make_async_copy(ones_v,
                                                  degacc.at[didx.at[sl1, i1]],
                                                  dsem).wait()
                    pltpu.async_copy(tbl.at[sidx.at[sl1, i1]],
                                     rowbuf[r1 % 2], gsem)
                    if i == 2 and gch >= 1 and gch + 1 < nchunks:
                        for d in idx_copies(gch + 1, (gch + 1) % 2):
                            d.start()
                pltpu.make_async_copy(tbl.at[sidx.at[sl, i]], rowbuf[b],
                                      gsem).wait()
                pltpu.async_copy(rowbuf[b], acc.at[didx.at[sl, i]], ssem,
                                 add=True)
                if dflag and dflag[r]:
                    pltpu.async_copy(ones_v, degacc.at[didx.at[sl, i]], dsem,
                                     add=True)
            for rr in (nr - 2, nr - 1):
                gch, i = divmod(rr, 8)
                pltpu.make_async_copy(rowbuf[rr % 2],
                                      acc.at[didx.at[gch % 2, i]],
                                      ssem).wait()
                if dflag and dflag[rr]:
                    pltpu.make_async_copy(ones_v,
                                          degacc.at[didx.at[gch % 2, i]],
                                          dsem).wait()

        if col_split:
            @pl.when(c == 0)
            def _():
                run(t0, s * nrow, dflags[0])

            @pl.when(c == 1)
            def _():
                run(t1, s * nrow, dflags[1])
        else:
            @pl.when(c == 0)
            def _():
                run(t0, s * (2 * nrow), None, nrow_n=64)

            @pl.when(c == 1)
            def _():
                run(t1, s * (2 * nrow) + 64, None, nrow_n=16)

        plsc.subcore_barrier()

        @pl.when(c == 0)
        def _():
            pltpu.sync_copy(acc.at[pl.ds(s * RPT, RPT), :],
                            o0.at[pl.ds(s * RPT, RPT), :])
            if col_split:
                pltpu.sync_copy(degacc.at[pl.ds(s * RPT, RPT)],
                                degoutA.at[pl.ds(s * RPT, RPT)])

        @pl.when(c == 1)
        def _():
            pltpu.sync_copy(acc.at[pl.ds(s * RPT, RPT), :],
                            o1.at[pl.ds(s * RPT, RPT), :])
            if col_split:
                pltpu.sync_copy(degacc.at[pl.ds(s * RPT, RPT)],
                                degoutB.at[pl.ds(s * RPT, RPT)])

    return pl.kernel(body, out_type=tuple(outs), mesh=mesh,
                     scratch_types=scratch)


_segsum_l1 = _make_segsum(True)
_segsum_l2 = _make_segsum(False)

_DN = (((1,), (1,)), ((), ()))


def _dot(a, w):
    return lax.dot_general(a, w, _DN, precision=lax.Precision.HIGHEST,
                           preferred_element_type=jnp.float32)


def _tc0_body(xr, w1r, out):
    out[...] = _dot(xr[...], w1r[...])


def _tc1_body(a0, a1, dA, dB, xw, w1l, b1r, w2l, ho, p2o, p2o2):
    inv = 1.0 / jnp.maximum(dA[...] + dB[...], 1.0)
    m0 = a0[...] * inv
    m1 = a1[...] * inv
    w1l_v = w1l[...]
    h = (_dot(m0, w1l_v[:, :128]) + _dot(m1, w1l_v[:, 128:])
         + xw[...] + b1r[...])
    h = jnp.maximum(h, 0.0)
    ho[...] = h
    p2 = _dot(h, w2l[...])
    p2o[...] = p2
    p2o2[...] = p2


def _tc1b_body(h, w2r, out):
    out[...] = _dot(h[...], w2r[...])


def _tc2_body(aA, aB, dA, dB, r2, b2r, out):
    inv = 1.0 / jnp.maximum(dA[...] + dB[...], 1.0)
    z = (aA[...] + aB[...]) * inv + b2r[...] + r2[...]
    m = jnp.max(z, axis=1, keepdims=True)
    lse = jnp.log(jnp.sum(jnp.exp(z - m), axis=1, keepdims=True))
    out[...] = z - m - lse


def _row_spec(d):
    return pl.BlockSpec((RB, d), lambda i: (i, 0))


def _full_spec(r, d):
    return pl.BlockSpec((r, d), lambda i: (0, 0))


_tc0 = pl.pallas_call(
    _tc0_body,
    grid=(GRID,),
    in_specs=[_row_spec(256), _full_spec(256, 256)],
    out_specs=_row_spec(256),
    out_shape=jax.ShapeDtypeStruct((N, 256), jnp.float32),
)

_tc1 = pl.pallas_call(
    _tc1_body,
    grid=(GRID,),
    in_specs=[_row_spec(128), _row_spec(128), _row_spec(1), _row_spec(1),
              _row_spec(256), _full_spec(256, 256), _full_spec(1, 256),
              _full_spec(128, 256)],
    out_specs=[_row_spec(256), _row_spec(128), _row_spec(128)],
    out_shape=[jax.ShapeDtypeStruct((N, 256), jnp.float32),
               jax.ShapeDtypeStruct((N, 128), jnp.float32),
               jax.ShapeDtypeStruct((N, 128), jnp.float32)],
)

_tc1b = pl.pallas_call(
    _tc1b_body,
    grid=(GRID,),
    in_specs=[_row_spec(256), _full_spec(128, 256)],
    out_specs=_row_spec(128),
    out_shape=jax.ShapeDtypeStruct((N, 128), jnp.float32),
)

_tc2 = pl.pallas_call(
    _tc2_body,
    grid=(GRID,),
    in_specs=[_row_spec(128), _row_spec(128), _row_spec(1), _row_spec(1),
              _row_spec(128), _full_spec(1, 128)],
    out_specs=_row_spec(128),
    out_shape=jax.ShapeDtypeStruct((N, 128), jnp.float32),
)


@jax.jit
def kernel(x, edge_index, W1_l, b1_l, W1_r, W2_l, b2_l, W2_r):
    src = edge_index[0].astype(jnp.int32)
    dst = edge_index[1].astype(jnp.int32)
    srcp = jnp.concatenate([src, jnp.zeros((EP - E,), jnp.int32)]).reshape(ER, 128)
    padd = N + (jnp.arange(EP - E, dtype=jnp.int32) % (NACC - N))
    dstp = jnp.concatenate([dst, padd]).reshape(ER, 128)
    x0 = x[:, :128]
    x1 = x[:, 128:]

    xw = _tc0(x, W1_r)
    agg0, agg1, degA, degB = _segsum_l1(x0, x1, srcp, dstp)
    dA = degA[:N].reshape(N, 1)
    dB = degB[:N].reshape(N, 1)
    h, p2a, p2b = _tc1(agg0, agg1, dA, dB, xw, W1_l, b1_l.reshape(1, -1),
                       W2_l)
    r2 = _tc1b(h, W2_r)
    a2A, a2B = _segsum_l2(p2a, p2b, srcp, dstp)
    return _tc2(a2A, a2B, dA, dB, r2, b2_l.reshape(1, -1))


# speedup vs baseline: 1.1888x; 1.0389x over previous
import jax
import jax.numpy as jnp
from jax import lax
from jax.experimental import pallas as pl
from jax.experimental.pallas import tpu as pltpu
from jax.experimental.pallas import tpu_sc as plsc

N = 10000
E = 160000
NC, NS, L = 2, 16, 16
ER = 1280
EP = ER * 128
NACC = 10240
RPT = NACC // NS
RB = 400
GRID = N // RB


def _make_segsum(col_split):
    mesh = plsc.VectorSubcoreMesh(core_axis_name="c", subcore_axis_name="s")
    outs = [jax.ShapeDtypeStruct((NACC, 128), jnp.float32),
            jax.ShapeDtypeStruct((NACC, 128), jnp.float32)]
    if col_split:
        outs += [jax.ShapeDtypeStruct((NACC,), jnp.float32),
                 jax.ShapeDtypeStruct((NACC,), jnp.float32)]
    scratch = [
        pltpu.VMEM_SHARED((NACC, 128), jnp.float32),
        pltpu.VMEM((2, 8, 128), jnp.int32),
        pltpu.VMEM((2, 8, 128), jnp.int32),
        pltpu.VMEM((128, 128), jnp.float32),
        pltpu.VMEM((128, 128), jnp.float32),
        pltpu.SemaphoreType.DMA,
        pltpu.SemaphoreType.DMA,
        pltpu.SemaphoreType.DMA,
        pltpu.SemaphoreType.DMA,
        pltpu.SemaphoreType.DMA,
    ]
    if col_split:
        scratch += [
            pltpu.VMEM_SHARED((NACC,), jnp.float32),
            pltpu.VMEM((128,), jnp.float32),
            pltpu.VMEM((RPT,), jnp.float32),
        ]

    def body(*refs):
        if col_split:
            (t0, t1, srcr, dstr, o0, o1, degoutA, degoutB,
             acc, sidx, didx, rows_a, rows_b,
             gsem, ssem, dsem, isem, zsem, degacc, ones_v, zdeg_v) = refs
        else:
            (t0, t1, srcr, dstr, o0, o1,
             acc, sidx, didx, rows_a, rows_b,
             gsem, ssem, dsem, isem, zsem) = refs
        c = lax.axis_index("c")
        s = lax.axis_index("s")
        rowbuf = [rows_a, rows_b]

        def zrow(i, carry):
            for j in range(128 // L):
                rows_a[i, pl.ds(j * L, L)] = jnp.zeros((L,), jnp.float32)
            return carry
        lax.fori_loop(0, 128, zrow, 0)
        for k in range(RPT // 128):
            pltpu.async_copy(rows_a, acc.at[pl.ds(s * RPT + k * 128, 128), :],
                             zsem)
        if col_split:
            def zdrow(i, carry):
                zdeg_v[pl.ds(i * L, L)] = jnp.zeros((L,), jnp.float32)
                return carry
            lax.fori_loop(0, RPT // L, zdrow, 0)
            for j in range(128 // L):
                ones_v[pl.ds(j * L, L)] = jnp.full((L,), 1.0, jnp.float32)
            pltpu.sync_copy(zdeg_v, degacc.at[pl.ds(s * RPT, RPT)])
        for k in range(RPT // 128):
            pltpu.make_async_copy(rows_a,
                                  acc.at[pl.ds(s * RPT + k * 128, 128), :],
                                  zsem).wait()
        plsc.subcore_barrier()

        if col_split:
            nrow = ER // NS
            row0 = s * nrow
            dflags = [[r < nrow // 2 for r in range(nrow)],
                      [r >= nrow // 2 for r in range(nrow)]]
        else:
            nrow = ER // NS // NC
            dflags = None

        def run(tbl, row0, dflag, nrow_n=None):
            nr = nrow if nrow_n is None else nrow_n
            nchunks = nr // 8

            def idx_copies(g, slot):
                base = row0 + g * 8
                return (pltpu.make_async_copy(srcr.at[pl.ds(base, 8), :],
                                              sidx.at[slot], isem),
                        pltpu.make_async_copy(dstr.at[pl.ds(base, 8), :],
                                              didx.at[slot], isem))

            pltpu.sync_copy(srcr.at[pl.ds(row0, 8), :], sidx.at[0])
            pltpu.sync_copy(dstr.at[pl.ds(row0, 8), :], didx.at[0])
            if nchunks > 1:
                for d in idx_copies(1, 1):
                    d.start()
            pltpu.async_copy(tbl.at[sidx.at[0, 0]], rows_a, gsem)
            for r in range(nr):
                gch, i = divmod(r, 8)
                sl = gch % 2
                b = r % 2
                r1 = r + 1
                if r1 < nr:
                    g1, i1 = divmod(r1, 8)
                    sl1 = g1 % 2
                    if i1 == 0:
                        for d in idx_copies(g1, sl1):
                            d.wait()
                    if r1 >= 2:
                        pltpu.make_async_copy(rowbuf[r1 % 2],
                                              acc.at[didx.at[sl1, i1]],
                                              ssem).wait()
                        if dflag and dflag[r - 1]:
                            pltpu.make_async_copy(ones_v,
                                                  degacc.at[didx.at[sl1, i1]],
                                                  dsem).wait()
                    pltpu.async_copy(tbl.at[sidx.at[sl1, i1]],
                                     rowbuf[r1 % 2], gsem)
                    if i == 2 and gch >= 1 and gch + 1 < nchunks:
                        for d in idx_copies(gch + 1, (gch + 1) % 2):
                            d.start()
                pltpu.make_async_copy(tbl.at[sidx.at[sl, i]], rowbuf[b],
                                      gsem).wait()
                pltpu.async_copy(rowbuf[b], acc.at[didx.at[sl, i]], ssem,
                                 add=True)
                if dflag and dflag[r]:
                    pltpu.async_copy(ones_v, degacc.at[didx.at[sl, i]], dsem,
                                     add=True)
            for rr in (nr - 2, nr - 1):
                gch, i = divmod(rr, 8)
                pltpu.make_async_copy(rowbuf[rr % 2],
                                      acc.at[didx.at[gch % 2, i]],
                                      ssem).wait()
                if dflag and dflag[rr]:
                    pltpu.make_async_copy(ones_v,
                                          degacc.at[didx.at[gch % 2, i]],
                                          dsem).wait()

        if col_split:
            @pl.when(c == 0)
            def _():
                run(t0, s * nrow, dflags[0])

            @pl.when(c == 1)
            def _():
                run(t1, s * nrow, dflags[1])
        else:
            @pl.when(c == 0)
            def _():
                run(t0, s * (2 * nrow), None, nrow_n=64)

            @pl.when(c == 1)
            def _():
                run(t1, s * (2 * nrow) + 64, None, nrow_n=16)

        plsc.subcore_barrier()

        @pl.when(c == 0)
        def _():
            pltpu.sync_copy(acc.at[pl.ds(s * RPT, RPT), :],
                            o0.at[pl.ds(s * RPT, RPT), :])
            if col_split:
                pltpu.sync_copy(degacc.at[pl.ds(s * RPT, RPT)],
                                degoutA.at[pl.ds(s * RPT, RPT)])

        @pl.when(c == 1)
        def _():
            pltpu.sync_copy(acc.at[pl.ds(s * RPT, RPT), :],
                            o1.at[pl.ds(s * RPT, RPT), :])
            if col_split:
                pltpu.sync_copy(degacc.at[pl.ds(s * RPT, RPT)],
                                degoutB.at[pl.ds(s * RPT, RPT)])

    return pl.kernel(body, out_type=tuple(outs), mesh=mesh,
                     scratch_types=scratch)


_segsum_l1 = _make_segsum(True)
_segsum_l2 = _make_segsum(False)

_DN = (((1,), (1,)), ((), ()))


def _dot(a, w):
    return lax.dot_general(a, w, _DN, precision=lax.Precision.DEFAULT,
                           preferred_element_type=jnp.float32)


def _tc0_body(xr, w1r, out):
    out[...] = _dot(xr[...], w1r[...])


def _tc1_body(a0, a1, dA, dB, xw, w1l, b1r, w2l, ho, p2o, p2o2):
    inv = 1.0 / jnp.maximum(dA[...] + dB[...], 1.0)
    m0 = a0[...] * inv
    m1 = a1[...] * inv
    w1l_v = w1l[...]
    h = (_dot(m0, w1l_v[:, :128]) + _dot(m1, w1l_v[:, 128:])
         + xw[...] + b1r[...])
    h = jnp.maximum(h, 0.0)
    ho[...] = h
    p2 = _dot(h, w2l[...])
    p2o[...] = p2
    p2o2[...] = p2


def _tc1b_body(h, w2r, out):
    out[...] = _dot(h[...], w2r[...])


def _tc2_body(aA, aB, dA, dB, r2, b2r, out):
    inv = 1.0 / jnp.maximum(dA[...] + dB[...], 1.0)
    z = (aA[...] + aB[...]) * inv + b2r[...] + r2[...]
    m = jnp.max(z, axis=1, keepdims=True)
    lse = jnp.log(jnp.sum(jnp.exp(z - m), axis=1, keepdims=True))
    out[...] = z - m - lse


def _row_spec(d):
    return pl.BlockSpec((RB, d), lambda i: (i, 0))


def _full_spec(r, d):
    return pl.BlockSpec((r, d), lambda i: (0, 0))


_tc0 = pl.pallas_call(
    _tc0_body,
    grid=(GRID,),
    in_specs=[_row_spec(256), _full_spec(256, 256)],
    out_specs=_row_spec(256),
    out_shape=jax.ShapeDtypeStruct((N, 256), jnp.float32),
)

_tc1 = pl.pallas_call(
    _tc1_body,
    grid=(GRID,),
    in_specs=[_row_spec(128), _row_spec(128), _row_spec(1), _row_spec(1),
              _row_spec(256), _full_spec(256, 256), _full_spec(1, 256),
              _full_spec(128, 256)],
    out_specs=[_row_spec(256), _row_spec(128), _row_spec(128)],
    out_shape=[jax.ShapeDtypeStruct((N, 256), jnp.float32),
               jax.ShapeDtypeStruct((N, 128), jnp.float32),
               jax.ShapeDtypeStruct((N, 128), jnp.float32)],
)

_tc1b = pl.pallas_call(
    _tc1b_body,
    grid=(GRID,),
    in_specs=[_row_spec(256), _full_spec(128, 256)],
    out_specs=_row_spec(128),
    out_shape=jax.ShapeDtypeStruct((N, 128), jnp.float32),
)

_tc2 = pl.pallas_call(
    _tc2_body,
    grid=(GRID,),
    in_specs=[_row_spec(128), _row_spec(128), _row_spec(1), _row_spec(1),
              _row_spec(128), _full_spec(1, 128)],
    out_specs=_row_spec(128),
    out_shape=jax.ShapeDtypeStruct((N, 128), jnp.float32),
)


@jax.jit
def kernel(x, edge_index, W1_l, b1_l, W1_r, W2_l, b2_l, W2_r):
    src = edge_index[0].astype(jnp.int32)
    dst = edge_index[1].astype(jnp.int32)
    srcp = jnp.concatenate([src, jnp.zeros((EP - E,), jnp.int32)]).reshape(ER, 128)
    padd = N + (jnp.arange(EP - E, dtype=jnp.int32) % (NACC - N))
    dstp = jnp.concatenate([dst, padd]).reshape(ER, 128)
    x0 = x[:, :128]
    x1 = x[:, 128:]

    xw = _tc0(x, W1_r)
    agg0, agg1, degA, degB = _segsum_l1(x0, x1, srcp, dstp)
    dA = degA[:N].reshape(N, 1)
    dB = degB[:N].reshape(N, 1)
    h, p2a, p2b = _tc1(agg0, agg1, dA, dB, xw, W1_l, b1_l.reshape(1, -1),
                       W2_l)
    r2 = _tc1b(h, W2_r)
    a2A, a2B = _segsum_l2(p2a, p2b, srcp, dstp)
    return _tc2(a2A, a2B, dA, dB, r2, b2_l.reshape(1, -1))
